# baseline (speedup 1.0000x reference)
import jax
import jax.numpy as jnp
from jax.experimental import pallas as pl
from jax.experimental.pallas import tpu as pltpu

B = 128
V = 100000
K = 64
NEG = -1e30


def _post_kernel(vals_ref, idx_ref, temp_ref, topk_ref, topp_ref, minp_ref,
                 g_ref, norm_ref, nid_ref):
    vals = vals_ref[...]
    idx = idx_ref[...]
    temp = temp_ref[...]
    tk = topk_ref[...]
    tp = topp_ref[...]
    mp_ = minp_ref[...]
    g = g_ref[...]

    s = vals / temp
    pos = jax.lax.broadcasted_iota(jnp.int32, (B, K), 1)
    k_mask = pos < tk
    masked = jnp.where(k_mask, s, NEG)
    mx = jax.lax.slice(masked, (0, 0), (B, 1))
    e = jnp.exp(masked - mx)
    z = jnp.sum(e, axis=1, keepdims=True)
    probs = e / z
    cum = probs
    sh = 1
    while sh < K:
        pad = jnp.zeros((B, sh), jnp.float32)
        cum = cum + jnp.concatenate(
            [pad, jax.lax.slice(cum, (0, 0), (B, K - sh))], axis=1)
        sh *= 2
    excl = cum - probs
    p_mask = excl < tp
    maxp = jax.lax.slice(probs, (0, 0), (B, 1))
    m_mask = probs >= mp_ * maxp
    final = (k_mask & p_mask & m_mask) | (pos == 0)
    mprobs = jnp.where(final, probs, 0.0)
    norm = mprobs / jnp.sum(mprobs, axis=1, keepdims=True)
    norm_ref[...] = norm

    score = jnp.where(final, masked + g, NEG)
    best = jnp.max(score, axis=1, keepdims=True)
    choice = jnp.min(jnp.where(score == best, pos, K), axis=1, keepdims=True)
    nid = jnp.sum(jnp.where(pos == choice, idx, 0), axis=1, keepdims=True)
    nid_ref[...] = nid


def kernel(logits, temperature, top_k, top_p, min_p):
    vals, idx = jax.lax.top_k(logits, K)
    g = jax.random.gumbel(jax.random.key(42), (B, K), jnp.float32)
    norm, nid = pl.pallas_call(
        _post_kernel,
        out_shape=(
            jax.ShapeDtypeStruct((B, K), jnp.float32),
            jax.ShapeDtypeStruct((B, 1), jnp.int32),
        ),
    )(vals, idx,
      temperature.reshape(B, 1), top_k.reshape(B, 1),
      top_p.reshape(B, 1), min_p.reshape(B, 1), g)
    return nid.reshape(B), norm


# speedup vs baseline: 6.7869x; 6.7869x over previous
import functools

import jax
import jax.numpy as jnp
from jax import lax
from jax.experimental import pallas as pl
from jax.experimental.pallas import tpu as pltpu
from jax.experimental.pallas import tpu_sc as plsc

B = 128
V = 100000
K = 64
NEG = -1e30

NW = 32
RPW = B // NW
CH = 20000
NCHUNK = V // CH
G = 10
GROUPS = CH // (16 * G)
DEPTH = 65
FCAP = 208


def _s(v, i):
    return lax.squeeze(lax.slice_in_dim(v, i, i + 1), (0,))


def _iota():
    return lax.iota(jnp.int32, 16)


def _f16(x):
    return jnp.full((16,), x, jnp.float32)


def _i16(x):
    return jnp.full((16,), x, jnp.int32)


def _sc_body(logits_hbm, scal_hbm, g_hbm, norm_hbm, nid_hbm,
             scal_v, g_v, chunk_v, fifo_v, fifo_p, bufv, bufi,
             tmp_f, tmp_i, norm_st, nid_st):
    wid = lax.axis_index("s") * 2 + lax.axis_index("c")
    pltpu.sync_copy(scal_hbm.at[wid], scal_v)
    pltpu.sync_copy(g_hbm.at[wid], g_v)

    def _popcnt(m):
        return _s(plsc.all_reduce_population_count(m), 0)

    def _vsum_f(x):
        for kk in (8, 4, 2, 1):
            tmp_f[...] = x
            x = x + plsc.load_gather(tmp_f, [_iota() ^ kk])
        return x

    def _vsum_i(x):
        for kk in (8, 4, 2, 1):
            tmp_i[...] = x
            x = x + plsc.load_gather(tmp_i, [_iota() ^ kk])
        return x

    def _prefix_f(x):
        for kk in (1, 2, 4, 8):
            tmp_f[...] = x
            sh = plsc.load_gather(
                tmp_f, [jnp.maximum(_iota() - kk, _i16(0))])
            x = x + jnp.where(_iota() >= _i16(kk), sh, _f16(0.0))
        return x

    def _minlane(x):
        sk, _sv = plsc.sort_key_val(x, _iota(), descending=False)
        return _s(sk, 0)

    def _maxlane(x):
        sk, _sv = plsc.sort_key_val(x, _iota(), descending=True)
        return _s(sk, 0)

    def bubble16(cv, cp):
        def bb(blk, s):
            cv, cp = s
            for u in range(8):
                j = blk * 8 + u
                bv = bufv[j]
                bi = bufi[j]
                m = cv > bv
                bufv[j] = jnp.where(m, cv, bv)
                bufi[j] = jnp.where(m, cp, bi)
                cv = jnp.where(m, bv, cv)
                cp = jnp.where(m, bi, cp)
            return (cv, cp)
        lax.fori_loop(0, K // 8, bb, (cv, cp))

    def row_body(r, nid_vec):
        row = wid * RPW + r

        def initb(j, c):
            bufv[j] = _f16(NEG)
            bufi[j] = _i16(0)
            return c
        lax.fori_loop(0, DEPTH, initb, 0)

        def chunk_body(c, carry):
            cnt, theta = carry
            pltpu.sync_copy(logits_hbm.at[pl.ds(row * V + c * CH, CH)],
                            chunk_v)

            def group_body(gi, carry):
                cnt, theta = carry
                base = gi * (16 * G)
                tvec = _f16(theta)
                vs = [chunk_v[pl.ds(base + 16 * j, 16)] for j in range(G)]
                gm = vs[0]
                for v in vs[1:]:
                    gm = jnp.maximum(gm, v)
                any_hit = _popcnt(gm >= tvec) > 0

                def trig(carry):
                    cnt, theta = carry
                    tv = _f16(theta)
                    cnt2 = cnt
                    for j in range(G):
                        m = vs[j] >= tv
                        plsc.store_compressed(
                            fifo_v.at[pl.ds(cnt2, 16)], vs[j], mask=m)
                        pos = _iota() + (c * CH + base + j * 16)
                        plsc.store_compressed(
                            fifo_p.at[pl.ds(cnt2, 16)], pos, mask=m)
                        cnt2 = cnt2 + _popcnt(m)

                    def dcond(s):
                        rd, th = s
                        return rd + 16 <= cnt2

                    def dbody(s):
                        rd, th = s
                        cv = fifo_v[pl.ds(rd, 16)]
                        cp = fifo_p[pl.ds(rd, 16)]
                        bubble16(cv, cp)
                        return (rd + 16, _minlane(bufv[3]))

                    rd, theta2 = lax.while_loop(
                        dcond, dbody, (jnp.int32(0), theta))
                    n = cnt2 - rd
                    mres = _iota() < _i16(n)
                    rv = plsc.load_expanded(
                        fifo_v.at[pl.ds(rd, 16)], mask=mres)
                    rp = plsc.load_expanded(
                        fifo_p.at[pl.ds(rd, 16)], mask=mres)
                    plsc.store_compressed(
                        fifo_v.at[pl.ds(0, 16)], rv, mask=mres)
                    plsc.store_compressed(
                        fifo_p.at[pl.ds(0, 16)], rp, mask=mres)
                    return (n, theta2)

                return lax.cond(any_hit, trig, lambda carry: carry,
                                (cnt, theta))

            return lax.fori_loop(0, GROUPS, group_body, (cnt, theta))

        cnt, theta = lax.fori_loop(
            0, NCHUNK, chunk_body, (jnp.int32(0), jnp.float32(NEG)))

        if True:
            mres = _iota() < _i16(cnt)
            cv = plsc.load_expanded(fifo_v.at[pl.ds(0, 16)], mask=mres)
            cp = plsc.load_expanded(fifo_p.at[pl.ds(0, 16)], mask=mres)
            bubble16(jnp.where(mres, cv, _f16(NEG)),
                     jnp.where(mres, cp, _i16(0)))

        def ext_body(k, carry):
            dv, ov, oi = carry
            W = plsc.load_gather(bufv, [dv, _iota()])
            Wi = plsc.load_gather(bufi, [dv, _iota()])
            packed = Wi * 16 + _iota()
            sv, sp = plsc.sort_key_val(W, packed, descending=True)
            p0 = _s(sp, 0)
            v0 = _f16(_s(sv, 0))
            tok = _i16(lax.shift_right_logical(p0, 4))
            l0 = _i16(lax.rem(p0, 16))
            dv = jnp.minimum(
                dv + jnp.where(_iota() == l0, _i16(1), _i16(0)),
                _i16(DEPTH - 1))
            ov = [jnp.where(_iota() == _i16(k - 16 * j), v0, ov[j])
                  for j in range(4)]
            oi = [jnp.where(_iota() == _i16(k - 16 * j), tok, oi[j])
                  for j in range(4)]
            return (dv, ov, oi)

        _, xs, ids = lax.fori_loop(
            0, K, ext_body,
            (_i16(0), [_f16(NEG)] * 4, [_i16(0)] * 4))

        tv_ = plsc.load_gather(scal_v, [_i16(r)])
        tkv = plsc.load_gather(scal_v, [_i16(RPW + r)])
        tpv = plsc.load_gather(scal_v, [_i16(2 * RPW + r)])
        mpv = plsc.load_gather(scal_v, [_i16(3 * RPW + r)])
        inv_t = _f16(1.0) / tv_
        pos = [_iota() + 16 * j for j in range(4)]
        posf = [p.astype(jnp.float32) for p in pos]
        km = [pf < tkv for pf in posf]
        masked = [jnp.where(km[j], xs[j] * inv_t, _f16(NEG))
                  for j in range(4)]
        m0v = _f16(_s(masked[0], 0))
        e = [jnp.exp(masked[j] - m0v) for j in range(4)]
        z = _vsum_f(e[0] + e[1] + e[2] + e[3])
        invz = _f16(1.0) / z
        pr = [e[j] * invz for j in range(4)]
        carry = _f16(0.0)
        excl = []
        for j in range(4):
            incl = _prefix_f(pr[j])
            excl.append(incl - pr[j] + carry)
            carry = carry + _f16(_s(incl, 15))
        pm = [excl[j] < tpv for j in range(4)]
        thr = mpv * _f16(_s(pr[0], 0))
        mm = [pr[j] >= thr for j in range(4)]
        fin = [(km[j] & pm[j] & mm[j]) | (pos[j] == _i16(0))
               for j in range(4)]
        mpr = [jnp.where(fin[j], pr[j], _f16(0.0)) for j in range(4)]
        zn = _vsum_f(mpr[0] + mpr[1] + mpr[2] + mpr[3])
        invzn = _f16(1.0) / zn
        for j in range(4):
            norm_st[pl.ds(r * K + 16 * j, 16)] = mpr[j] * invzn
        gs = [g_v[pl.ds(r * K + 16 * j, 16)] for j in range(4)]
        sc = [jnp.where(fin[j], masked[j] + gs[j], _f16(NEG))
              for j in range(4)]
        sv = jnp.maximum(jnp.maximum(sc[0], sc[1]),
                         jnp.maximum(sc[2], sc[3]))
        bms = _f16(_maxlane(sv))
        ch = [jnp.where(sc[j] == bms, pos[j], _i16(K)) for j in range(4)]
        cv2 = jnp.minimum(jnp.minimum(ch[0], ch[1]),
                          jnp.minimum(ch[2], ch[3]))
        sk, _sv2 = plsc.sort_key_val(cv2, _iota(), descending=False)
        choice = _i16(_s(sk, 0))
        nsel = [jnp.where(pos[j] == choice, ids[j], _i16(0))
                for j in range(4)]
        nid_spl = _vsum_i(nsel[0] + nsel[1] + nsel[2] + nsel[3])
        return jnp.where(_iota() == _i16(r), nid_spl, nid_vec)

    nid_vec_out = lax.fori_loop(0, RPW, row_body, _i16(0))
    nid_st[...] = nid_vec_out
    pltpu.sync_copy(norm_st, norm_hbm.at[pl.ds(wid * RPW * K, RPW * K)])
    pltpu.sync_copy(nid_st, nid_hbm.at[wid])


@functools.partial(jax.jit, static_argnums=())
def _sc_call(logits1d, scal, g):
    mesh = plsc.VectorSubcoreMesh(core_axis_name="c", subcore_axis_name="s")
    f = pl.kernel(
        _sc_body,
        mesh=mesh,
        compiler_params=pltpu.CompilerParams(needs_layout_passes=False),
        out_type=(
            jax.ShapeDtypeStruct((B * K,), jnp.float32),
            jax.ShapeDtypeStruct((NW, 16), jnp.int32),
        ),
        scratch_types=[
            pltpu.VMEM((16,), jnp.float32),
            pltpu.VMEM((RPW * K,), jnp.float32),
            pltpu.VMEM((CH,), jnp.float32),
            pltpu.VMEM((FCAP,), jnp.float32),
            pltpu.VMEM((FCAP,), jnp.int32),
            pltpu.VMEM((DEPTH, 16), jnp.float32),
            pltpu.VMEM((DEPTH, 16), jnp.int32),
            pltpu.VMEM((16,), jnp.float32),
            pltpu.VMEM((16,), jnp.int32),
            pltpu.VMEM((RPW * K,), jnp.float32),
            pltpu.VMEM((16,), jnp.int32),
        ],
    )
    return f(logits1d, scal, g)


def kernel(logits, temperature, top_k, top_p, min_p):
    logits1d = logits.reshape(B * V)
    scal = jnp.concatenate(
        [temperature.reshape(NW, RPW),
         top_k.astype(jnp.float32).reshape(NW, RPW),
         top_p.reshape(NW, RPW),
         min_p.reshape(NW, RPW)], axis=1)
    g = jax.random.gumbel(
        jax.random.key(42), (B, K), jnp.float32).reshape(NW, RPW * K)
    norm1d, nid2 = _sc_call(logits1d, scal, g)
    return nid2[:, :RPW].reshape(B), norm1d.reshape(B, K)


# speedup vs baseline: 7.7980x; 1.1490x over previous
import functools

import jax
import jax.numpy as jnp
from jax import lax
from jax.experimental import pallas as pl
from jax.experimental.pallas import tpu as pltpu
from jax.experimental.pallas import tpu_sc as plsc

B = 128
V = 100000
K = 64
NEG = -1e30

NW = 32
RPW = B // NW
CH = 50000
NCHUNK = V // CH
G = 25
GROUPS = CH // (16 * G)
DEPTH = 65
FCAP = 448


def _s(v, i):
    return lax.squeeze(lax.slice_in_dim(v, i, i + 1), (0,))


def _iota():
    return lax.iota(jnp.int32, 16)


def _f16(x):
    return jnp.full((16,), x, jnp.float32)


def _i16(x):
    return jnp.full((16,), x, jnp.int32)


def _sc_body(logits_hbm, scal_hbm, g_hbm, norm_hbm, nid_hbm,
             scal_v, g_v, chunk_v, fifo_v, fifo_p, bufv, bufi,
             tmp_f, tmp_i, norm_st, nid_st):
    wid = lax.axis_index("s") * 2 + lax.axis_index("c")
    pltpu.sync_copy(scal_hbm.at[wid], scal_v)
    pltpu.sync_copy(g_hbm.at[wid], g_v)

    def _popcnt(m):
        return _s(plsc.all_reduce_population_count(m), 0)

    def _vsum_f(x):
        for kk in (8, 4, 2, 1):
            tmp_f[...] = x
            x = x + plsc.load_gather(tmp_f, [_iota() ^ kk])
        return x

    def _vsum_i(x):
        for kk in (8, 4, 2, 1):
            tmp_i[...] = x
            x = x + plsc.load_gather(tmp_i, [_iota() ^ kk])
        return x

    def _prefix_f(x):
        for kk in (1, 2, 4, 8):
            tmp_f[...] = x
            sh = plsc.load_gather(
                tmp_f, [jnp.maximum(_iota() - kk, _i16(0))])
            x = x + jnp.where(_iota() >= _i16(kk), sh, _f16(0.0))
        return x

    def _minlane(x):
        sk, _sv = plsc.sort_key_val(x, _iota(), descending=False)
        return _s(sk, 0)

    def _maxlane(x):
        sk, _sv = plsc.sort_key_val(x, _iota(), descending=True)
        return _s(sk, 0)

    def bubble16(cv, cp):
        def bb(blk, s):
            cv, cp = s
            for u in range(8):
                j = blk * 8 + u
                bv = bufv[j]
                bi = bufi[j]
                m = cv > bv
                bufv[j] = jnp.where(m, cv, bv)
                bufi[j] = jnp.where(m, cp, bi)
                cv = jnp.where(m, bv, cv)
                cp = jnp.where(m, bi, cp)
            return (cv, cp)
        lax.fori_loop(0, K // 8, bb, (cv, cp))

    def row_body(r, nid_vec):
        row = wid * RPW + r

        def initb(j, c):
            bufv[j] = _f16(NEG)
            bufi[j] = _i16(0)
            return c
        lax.fori_loop(0, DEPTH, initb, 0)

        def chunk_body(c, carry):
            cnt, theta = carry
            pltpu.sync_copy(logits_hbm.at[pl.ds(row * V + c * CH, CH)],
                            chunk_v)

            def group_body(gi, carry):
                cnt, theta = carry
                base = gi * (16 * G)
                tvec = _f16(theta)
                vs = [chunk_v[pl.ds(base + 16 * j, 16)] for j in range(G)]
                gm = vs[0]
                for v in vs[1:]:
                    gm = jnp.maximum(gm, v)
                any_hit = _popcnt(gm >= tvec) > 0

                def trig(carry):
                    cnt, theta = carry
                    tv = _f16(theta)
                    cnt2 = cnt
                    for j in range(G):
                        m = vs[j] >= tv
                        plsc.store_compressed(
                            fifo_v.at[pl.ds(cnt2, 16)], vs[j], mask=m)
                        pos = _iota() + (c * CH + base + j * 16)
                        plsc.store_compressed(
                            fifo_p.at[pl.ds(cnt2, 16)], pos, mask=m)
                        cnt2 = cnt2 + _popcnt(m)

                    def dcond(s):
                        rd, th = s
                        return rd + 16 <= cnt2

                    def dbody(s):
                        rd, th = s
                        cv = fifo_v[pl.ds(rd, 16)]
                        cp = fifo_p[pl.ds(rd, 16)]
                        bubble16(cv, cp)
                        return (rd + 16, _minlane(bufv[3]))

                    rd, theta2 = lax.while_loop(
                        dcond, dbody, (jnp.int32(0), theta))
                    n = cnt2 - rd
                    mres = _iota() < _i16(n)
                    rv = plsc.load_expanded(
                        fifo_v.at[pl.ds(rd, 16)], mask=mres)
                    rp = plsc.load_expanded(
                        fifo_p.at[pl.ds(rd, 16)], mask=mres)
                    plsc.store_compressed(
                        fifo_v.at[pl.ds(0, 16)], rv, mask=mres)
                    plsc.store_compressed(
                        fifo_p.at[pl.ds(0, 16)], rp, mask=mres)
                    return (n, theta2)

                return lax.cond(any_hit, trig, lambda carry: carry,
                                (cnt, theta))

            return lax.fori_loop(0, GROUPS, group_body, (cnt, theta))

        cnt, theta = lax.fori_loop(
            0, NCHUNK, chunk_body, (jnp.int32(0), jnp.float32(NEG)))

        if True:
            mres = _iota() < _i16(cnt)
            cv = plsc.load_expanded(fifo_v.at[pl.ds(0, 16)], mask=mres)
            cp = plsc.load_expanded(fifo_p.at[pl.ds(0, 16)], mask=mres)
            bubble16(jnp.where(mres, cv, _f16(NEG)),
                     jnp.where(mres, cp, _i16(0)))

        def ext_body(k, carry):
            dv, ov, oi = carry
            W = plsc.load_gather(bufv, [dv, _iota()])
            Wi = plsc.load_gather(bufi, [dv, _iota()])
            packed = Wi * 16 + _iota()
            sv, sp = plsc.sort_key_val(W, packed, descending=True)
            p0 = _s(sp, 0)
            v0 = _f16(_s(sv, 0))
            tok = _i16(lax.shift_right_logical(p0, 4))
            l0 = _i16(lax.rem(p0, 16))
            dv = jnp.minimum(
                dv + jnp.where(_iota() == l0, _i16(1), _i16(0)),
                _i16(DEPTH - 1))
            ov = [jnp.where(_iota() == _i16(k - 16 * j), v0, ov[j])
                  for j in range(4)]
            oi = [jnp.where(_iota() == _i16(k - 16 * j), tok, oi[j])
                  for j in range(4)]
            return (dv, ov, oi)

        _, xs, ids = lax.fori_loop(
            0, K, ext_body,
            (_i16(0), [_f16(NEG)] * 4, [_i16(0)] * 4))

        tv_ = plsc.load_gather(scal_v, [_i16(r)])
        tkv = plsc.load_gather(scal_v, [_i16(RPW + r)])
        tpv = plsc.load_gather(scal_v, [_i16(2 * RPW + r)])
        mpv = plsc.load_gather(scal_v, [_i16(3 * RPW + r)])
        inv_t = _f16(1.0) / tv_
        pos = [_iota() + 16 * j for j in range(4)]
        posf = [p.astype(jnp.float32) for p in pos]
        km = [pf < tkv for pf in posf]
        masked = [jnp.where(km[j], xs[j] * inv_t, _f16(NEG))
                  for j in range(4)]
        m0v = _f16(_s(masked[0], 0))
        e = [jnp.exp(masked[j] - m0v) for j in range(4)]
        z = _vsum_f(e[0] + e[1] + e[2] + e[3])
        invz = _f16(1.0) / z
        pr = [e[j] * invz for j in range(4)]
        carry = _f16(0.0)
        excl = []
        for j in range(4):
            incl = _prefix_f(pr[j])
            excl.append(incl - pr[j] + carry)
            carry = carry + _f16(_s(incl, 15))
        pm = [excl[j] < tpv for j in range(4)]
        thr = mpv * _f16(_s(pr[0], 0))
        mm = [pr[j] >= thr for j in range(4)]
        fin = [(km[j] & pm[j] & mm[j]) | (pos[j] == _i16(0))
               for j in range(4)]
        mpr = [jnp.where(fin[j], pr[j], _f16(0.0)) for j in range(4)]
        zn = _vsum_f(mpr[0] + mpr[1] + mpr[2] + mpr[3])
        invzn = _f16(1.0) / zn
        for j in range(4):
            norm_st[pl.ds(r * K + 16 * j, 16)] = mpr[j] * invzn
        gs = [g_v[pl.ds(r * K + 16 * j, 16)] for j in range(4)]
        sc = [jnp.where(fin[j], masked[j] + gs[j], _f16(NEG))
              for j in range(4)]
        sv = jnp.maximum(jnp.maximum(sc[0], sc[1]),
                         jnp.maximum(sc[2], sc[3]))
        bms = _f16(_maxlane(sv))
        ch = [jnp.where(sc[j] == bms, pos[j], _i16(K)) for j in range(4)]
        cv2 = jnp.minimum(jnp.minimum(ch[0], ch[1]),
                          jnp.minimum(ch[2], ch[3]))
        sk, _sv2 = plsc.sort_key_val(cv2, _iota(), descending=False)
        choice = _i16(_s(sk, 0))
        nsel = [jnp.where(pos[j] == choice, ids[j], _i16(0))
                for j in range(4)]
        nid_spl = _vsum_i(nsel[0] + nsel[1] + nsel[2] + nsel[3])
        return jnp.where(_iota() == _i16(r), nid_spl, nid_vec)

    nid_vec_out = lax.fori_loop(0, RPW, row_body, _i16(0))
    nid_st[...] = nid_vec_out
    pltpu.sync_copy(norm_st, norm_hbm.at[pl.ds(wid * RPW * K, RPW * K)])
    pltpu.sync_copy(nid_st, nid_hbm.at[wid])


@functools.partial(jax.jit, static_argnums=())
def _sc_call(logits1d, scal, g):
    mesh = plsc.VectorSubcoreMesh(core_axis_name="c", subcore_axis_name="s")
    f = pl.kernel(
        _sc_body,
        mesh=mesh,
        compiler_params=pltpu.CompilerParams(needs_layout_passes=False),
        out_type=(
            jax.ShapeDtypeStruct((B * K,), jnp.float32),
            jax.ShapeDtypeStruct((NW, 16), jnp.int32),
        ),
        scratch_types=[
            pltpu.VMEM((16,), jnp.float32),
            pltpu.VMEM((RPW * K,), jnp.float32),
            pltpu.VMEM((CH,), jnp.float32),
            pltpu.VMEM((FCAP,), jnp.float32),
            pltpu.VMEM((FCAP,), jnp.int32),
            pltpu.VMEM((DEPTH, 16), jnp.float32),
            pltpu.VMEM((DEPTH, 16), jnp.int32),
            pltpu.VMEM((16,), jnp.float32),
            pltpu.VMEM((16,), jnp.int32),
            pltpu.VMEM((RPW * K,), jnp.float32),
            pltpu.VMEM((16,), jnp.int32),
        ],
    )
    return f(logits1d, scal, g)


def kernel(logits, temperature, top_k, top_p, min_p):
    logits1d = logits.reshape(B * V)
    scal = jnp.concatenate(
        [temperature.reshape(NW, RPW),
         top_k.astype(jnp.float32).reshape(NW, RPW),
         top_p.reshape(NW, RPW),
         min_p.reshape(NW, RPW)], axis=1)
    g = jax.random.gumbel(
        jax.random.key(42), (B, K), jnp.float32).reshape(NW, RPW * K)
    norm1d, nid2 = _sc_call(logits1d, scal, g)
    return nid2[:, :RPW].reshape(B), norm1d.reshape(B, K)


# speedup vs baseline: 8.2193x; 1.0540x over previous
import functools

import jax
import jax.numpy as jnp
from jax import lax
from jax.experimental import pallas as pl
from jax.experimental.pallas import tpu as pltpu
from jax.experimental.pallas import tpu_sc as plsc

B = 128
V = 100000
K = 64
NEG = -1e30

NW = 32
RPW = B // NW
CH = 50000
NCHUNK = V // CH
G = 25
GROUPS = CH // (16 * G)
DEPTH = 65
FCAP = 448


def _s(v, i):
    return lax.squeeze(lax.slice_in_dim(v, i, i + 1), (0,))


def _iota():
    return lax.iota(jnp.int32, 16)


def _f16(x):
    return jnp.full((16,), x, jnp.float32)


def _i16(x):
    return jnp.full((16,), x, jnp.int32)


def _sc_body(logits_hbm, scal_hbm, g_hbm, norm_hbm, nid_hbm,
             scal_v, g_v, chunk_v, fifo_v, fifo_p, bufv, bufi,
             tmp_f, tmp_i, norm_st, nid_st, dma_sem):
    wid = lax.axis_index("s") * 2 + lax.axis_index("c")
    wbase = wid * (RPW * V)
    pltpu.sync_copy(scal_hbm.at[wid], scal_v)
    pltpu.sync_copy(g_hbm.at[wid], g_v)
    pltpu.async_copy(logits_hbm.at[pl.ds(wbase, CH)],
                     chunk_v.at[pl.ds(0, CH)], dma_sem)

    def _popcnt(m):
        return _s(plsc.all_reduce_population_count(m), 0)

    def _vsum_f(x):
        for kk in (8, 4, 2, 1):
            tmp_f[...] = x
            x = x + plsc.load_gather(tmp_f, [_iota() ^ kk])
        return x

    def _vsum_i(x):
        for kk in (8, 4, 2, 1):
            tmp_i[...] = x
            x = x + plsc.load_gather(tmp_i, [_iota() ^ kk])
        return x

    def _prefix_f(x):
        for kk in (1, 2, 4, 8):
            tmp_f[...] = x
            sh = plsc.load_gather(
                tmp_f, [jnp.maximum(_iota() - kk, _i16(0))])
            x = x + jnp.where(_iota() >= _i16(kk), sh, _f16(0.0))
        return x

    def _minlane(x):
        sk, _sv = plsc.sort_key_val(x, _iota(), descending=False)
        return _s(sk, 0)

    def _maxlane(x):
        sk, _sv = plsc.sort_key_val(x, _iota(), descending=True)
        return _s(sk, 0)

    def bubble16(cv, cp):
        def bb(blk, s):
            cv, cp = s
            for u in range(8):
                j = blk * 8 + u
                bv = bufv[j]
                bi = bufi[j]
                m = cv > bv
                bufv[j] = jnp.where(m, cv, bv)
                bufi[j] = jnp.where(m, cp, bi)
                cv = jnp.where(m, bv, cv)
                cp = jnp.where(m, bi, cp)
            return (cv, cp)
        lax.fori_loop(0, K // 8, bb, (cv, cp))

    def row_body(r, nid_vec):
        row = wid * RPW + r

        def initb(j, c):
            bufv[j] = _f16(NEG)
            bufi[j] = _i16(0)
            return c
        lax.fori_loop(0, DEPTH, initb, 0)

        def chunk_body(c, carry):
            cnt, theta = carry
            seg = r * NCHUNK + c
            cb = lax.rem(seg, 2)
            pltpu.make_async_copy(
                logits_hbm.at[pl.ds(wbase, CH)],
                chunk_v.at[pl.ds(cb * CH, CH)], dma_sem).wait()
            nxt = seg + 1

            @pl.when(nxt < RPW * NCHUNK)
            def _prefetch():
                pltpu.async_copy(
                    logits_hbm.at[pl.ds(wbase + nxt * CH, CH)],
                    chunk_v.at[pl.ds(lax.rem(nxt, 2) * CH, CH)], dma_sem)

            def group_body(gi, carry):
                cnt, theta = carry
                base = gi * (16 * G)
                tvec = _f16(theta)
                vs = [chunk_v[pl.ds(cb * CH + base + 16 * j, 16)]
                      for j in range(G)]
                gm = vs[0]
                for v in vs[1:]:
                    gm = jnp.maximum(gm, v)
                any_hit = _popcnt(gm >= tvec) > 0

                def trig(carry):
                    cnt, theta = carry
                    tv = _f16(theta)
                    cnt2 = cnt
                    for j in range(G):
                        m = vs[j] >= tv
                        plsc.store_compressed(
                            fifo_v.at[pl.ds(cnt2, 16)], vs[j], mask=m)
                        pos = _iota() + (c * CH + base + j * 16)
                        plsc.store_compressed(
                            fifo_p.at[pl.ds(cnt2, 16)], pos, mask=m)
                        cnt2 = cnt2 + _popcnt(m)

                    def dcond(s):
                        rd, th = s
                        return rd + 16 <= cnt2

                    def dbody(s):
                        rd, th = s
                        cv = fifo_v[pl.ds(rd, 16)]
                        cp = fifo_p[pl.ds(rd, 16)]
                        bubble16(cv, cp)
                        return (rd + 16, _minlane(bufv[3]))

                    rd, theta2 = lax.while_loop(
                        dcond, dbody, (jnp.int32(0), theta))
                    n = cnt2 - rd
                    mres = _iota() < _i16(n)
                    rv = plsc.load_expanded(
                        fifo_v.at[pl.ds(rd, 16)], mask=mres)
                    rp = plsc.load_expanded(
                        fifo_p.at[pl.ds(rd, 16)], mask=mres)
                    plsc.store_compressed(
                        fifo_v.at[pl.ds(0, 16)], rv, mask=mres)
                    plsc.store_compressed(
                        fifo_p.at[pl.ds(0, 16)], rp, mask=mres)
                    return (n, theta2)

                return lax.cond(any_hit, trig, lambda carry: carry,
                                (cnt, theta))

            return lax.fori_loop(0, GROUPS, group_body, (cnt, theta))

        cnt, theta = lax.fori_loop(
            0, NCHUNK, chunk_body, (jnp.int32(0), jnp.float32(NEG)))

        if True:
            mres = _iota() < _i16(cnt)
            cv = plsc.load_expanded(fifo_v.at[pl.ds(0, 16)], mask=mres)
            cp = plsc.load_expanded(fifo_p.at[pl.ds(0, 16)], mask=mres)
            bubble16(jnp.where(mres, cv, _f16(NEG)),
                     jnp.where(mres, cp, _i16(0)))

        def ext_body(k, carry):
            dv, ov, oi = carry
            W = plsc.load_gather(bufv, [dv, _iota()])
            Wi = plsc.load_gather(bufi, [dv, _iota()])
            packed = Wi * 16 + _iota()
            sv, sp = plsc.sort_key_val(W, packed, descending=True)
            p0 = _s(sp, 0)
            v0 = _f16(_s(sv, 0))
            tok = _i16(lax.shift_right_logical(p0, 4))
            l0 = _i16(lax.rem(p0, 16))
            dv = jnp.minimum(
                dv + jnp.where(_iota() == l0, _i16(1), _i16(0)),
                _i16(DEPTH - 1))
            ov = [jnp.where(_iota() == _i16(k - 16 * j), v0, ov[j])
                  for j in range(4)]
            oi = [jnp.where(_iota() == _i16(k - 16 * j), tok, oi[j])
                  for j in range(4)]
            return (dv, ov, oi)

        _, xs, ids = lax.fori_loop(
            0, K, ext_body,
            (_i16(0), [_f16(NEG)] * 4, [_i16(0)] * 4))

        tv_ = plsc.load_gather(scal_v, [_i16(r)])
        tkv = plsc.load_gather(scal_v, [_i16(RPW + r)])
        tpv = plsc.load_gather(scal_v, [_i16(2 * RPW + r)])
        mpv = plsc.load_gather(scal_v, [_i16(3 * RPW + r)])
        inv_t = _f16(1.0) / tv_
        pos = [_iota() + 16 * j for j in range(4)]
        posf = [p.astype(jnp.float32) for p in pos]
        km = [pf < tkv for pf in posf]
        masked = [jnp.where(km[j], xs[j] * inv_t, _f16(NEG))
                  for j in range(4)]
        m0v = _f16(_s(masked[0], 0))
        e = [jnp.exp(masked[j] - m0v) for j in range(4)]
        z = _vsum_f(e[0] + e[1] + e[2] + e[3])
        invz = _f16(1.0) / z
        pr = [e[j] * invz for j in range(4)]
        carry = _f16(0.0)
        excl = []
        for j in range(4):
            incl = _prefix_f(pr[j])
            excl.append(incl - pr[j] + carry)
            carry = carry + _f16(_s(incl, 15))
        pm = [excl[j] < tpv for j in range(4)]
        thr = mpv * _f16(_s(pr[0], 0))
        mm = [pr[j] >= thr for j in range(4)]
        fin = [(km[j] & pm[j] & mm[j]) | (pos[j] == _i16(0))
               for j in range(4)]
        mpr = [jnp.where(fin[j], pr[j], _f16(0.0)) for j in range(4)]
        zn = _vsum_f(mpr[0] + mpr[1] + mpr[2] + mpr[3])
        invzn = _f16(1.0) / zn
        for j in range(4):
            norm_st[pl.ds(r * K + 16 * j, 16)] = mpr[j] * invzn
        gs = [g_v[pl.ds(r * K + 16 * j, 16)] for j in range(4)]
        sc = [jnp.where(fin[j], masked[j] + gs[j], _f16(NEG))
              for j in range(4)]
        sv = jnp.maximum(jnp.maximum(sc[0], sc[1]),
                         jnp.maximum(sc[2], sc[3]))
        bms = _f16(_maxlane(sv))
        ch = [jnp.where(sc[j] == bms, pos[j], _i16(K)) for j in range(4)]
        cv2 = jnp.minimum(jnp.minimum(ch[0], ch[1]),
                          jnp.minimum(ch[2], ch[3]))
        sk, _sv2 = plsc.sort_key_val(cv2, _iota(), descending=False)
        choice = _i16(_s(sk, 0))
        nsel = [jnp.where(pos[j] == choice, ids[j], _i16(0))
                for j in range(4)]
        nid_spl = _vsum_i(nsel[0] + nsel[1] + nsel[2] + nsel[3])
        return jnp.where(_iota() == _i16(r), nid_spl, nid_vec)

    nid_vec_out = lax.fori_loop(0, RPW, row_body, _i16(0))
    nid_st[...] = nid_vec_out
    pltpu.sync_copy(norm_st, norm_hbm.at[pl.ds(wid * RPW * K, RPW * K)])
    pltpu.sync_copy(nid_st, nid_hbm.at[wid])


@functools.partial(jax.jit, static_argnums=())
def _sc_call(logits1d, scal, g):
    mesh = plsc.VectorSubcoreMesh(core_axis_name="c", subcore_axis_name="s")
    f = pl.kernel(
        _sc_body,
        mesh=mesh,
        compiler_params=pltpu.CompilerParams(needs_layout_passes=False),
        out_type=(
            jax.ShapeDtypeStruct((B * K,), jnp.float32),
            jax.ShapeDtypeStruct((NW, 16), jnp.int32),
        ),
        scratch_types=[
            pltpu.VMEM((16,), jnp.float32),
            pltpu.VMEM((RPW * K,), jnp.float32),
            pltpu.VMEM((2 * CH,), jnp.float32),
            pltpu.VMEM((FCAP,), jnp.float32),
            pltpu.VMEM((FCAP,), jnp.int32),
            pltpu.VMEM((DEPTH, 16), jnp.float32),
            pltpu.VMEM((DEPTH, 16), jnp.int32),
            pltpu.VMEM((16,), jnp.float32),
            pltpu.VMEM((16,), jnp.int32),
            pltpu.VMEM((RPW * K,), jnp.float32),
            pltpu.VMEM((16,), jnp.int32),
            pltpu.SemaphoreType.DMA,
        ],
    )
    return f(logits1d, scal, g)


def kernel(logits, temperature, top_k, top_p, min_p):
    logits1d = logits.reshape(B * V)
    scal = jnp.concatenate(
        [temperature.reshape(NW, RPW),
         top_k.astype(jnp.float32).reshape(NW, RPW),
         top_p.reshape(NW, RPW),
         min_p.reshape(NW, RPW)], axis=1)
    g = jax.random.gumbel(
        jax.random.key(42), (B, K), jnp.float32).reshape(NW, RPW * K)
    norm1d, nid2 = _sc_call(logits1d, scal, g)
    return nid2[:, :RPW].reshape(B), norm1d.reshape(B, K)


# speedup vs baseline: 8.2589x; 1.0048x over previous
import functools

import jax
import jax.numpy as jnp
from jax import lax
from jax.experimental import pallas as pl
from jax.experimental.pallas import tpu as pltpu
from jax.experimental.pallas import tpu_sc as plsc

B = 128
V = 100000
K = 64
NEG = -1e30

NW = 32
RPW = B // NW
CH = 50000
NCHUNK = V // CH
G = 25
GROUPS = CH // (16 * G)
DEPTH = 65
FCAP = 448


def _s(v, i):
    return lax.squeeze(lax.slice_in_dim(v, i, i + 1), (0,))


def _iota():
    return lax.iota(jnp.int32, 16)


def _f16(x):
    return jnp.full((16,), x, jnp.float32)


def _i16(x):
    return jnp.full((16,), x, jnp.int32)


def _sc_body(logits_hbm, scal_hbm, g_hbm, norm_hbm, nid_hbm,
             scal_v, g_v, chunk_v, fifo_v, fifo_p, bufv, bufi,
             tmp_f, tmp_i, norm_st, nid_st, dma_sem0, dma_sem1):
    wid = lax.axis_index("s") * 2 + lax.axis_index("c")
    wbase = wid * (RPW * V)
    pltpu.sync_copy(scal_hbm.at[wid], scal_v)
    pltpu.sync_copy(g_hbm.at[wid], g_v)
    pltpu.async_copy(logits_hbm.at[pl.ds(wbase, CH)],
                     chunk_v.at[pl.ds(0, CH)], dma_sem0)

    def _popcnt(m):
        return _s(plsc.all_reduce_population_count(m), 0)

    def _vsum_f(x):
        for kk in (8, 4, 2, 1):
            tmp_f[...] = x
            x = x + plsc.load_gather(tmp_f, [_iota() ^ kk])
        return x

    def _vsum_i(x):
        for kk in (8, 4, 2, 1):
            tmp_i[...] = x
            x = x + plsc.load_gather(tmp_i, [_iota() ^ kk])
        return x

    def _prefix_f(x):
        for kk in (1, 2, 4, 8):
            tmp_f[...] = x
            sh = plsc.load_gather(
                tmp_f, [jnp.maximum(_iota() - kk, _i16(0))])
            x = x + jnp.where(_iota() >= _i16(kk), sh, _f16(0.0))
        return x

    def _minlane(x):
        sk, _sv = plsc.sort_key_val(x, _iota(), descending=False)
        return _s(sk, 0)

    def _maxlane(x):
        sk, _sv = plsc.sort_key_val(x, _iota(), descending=True)
        return _s(sk, 0)

    def bubble16(cv, cp):
        def bb(blk, s):
            cv, cp = s
            for u in range(8):
                j = blk * 8 + u
                bv = bufv[j]
                bi = bufi[j]
                m = cv > bv
                bufv[j] = jnp.where(m, cv, bv)
                bufi[j] = jnp.where(m, cp, bi)
                cv = jnp.where(m, bv, cv)
                cp = jnp.where(m, bi, cp)
            return (cv, cp)
        lax.fori_loop(0, K // 8, bb, (cv, cp))

    def row_body(r, nid_vec):
        row = wid * RPW + r

        def initb(j, c):
            bufv[j] = _f16(NEG)
            bufi[j] = _i16(0)
            return c
        lax.fori_loop(0, DEPTH, initb, 0)

        def chunk_body(c, carry, cb, sem_cur, sem_nxt):
            cnt, theta = carry
            seg = r * NCHUNK + c
            pltpu.make_async_copy(
                logits_hbm.at[pl.ds(wbase, CH)],
                chunk_v.at[pl.ds(cb * CH, CH)], sem_cur).wait()
            nxt = seg + 1

            @pl.when(nxt < RPW * NCHUNK)
            def _prefetch():
                pltpu.async_copy(
                    logits_hbm.at[pl.ds(wbase + nxt * CH, CH)],
                    chunk_v.at[pl.ds(((cb + 1) % 2) * CH, CH)], sem_nxt)

            def group_body(gi, carry):
                cnt, theta = carry
                base = gi * (16 * G)
                tvec = _f16(theta)
                vs = [chunk_v[pl.ds(cb * CH + base + 16 * j, 16)]
                      for j in range(G)]
                gm = vs[0]
                for v in vs[1:]:
                    gm = jnp.maximum(gm, v)
                any_hit = _popcnt(gm >= tvec) > 0

                def trig(carry):
                    cnt, theta = carry
                    tv = _f16(theta)
                    cnt2 = cnt
                    for j in range(G):
                        m = vs[j] >= tv
                        plsc.store_compressed(
                            fifo_v.at[pl.ds(cnt2, 16)], vs[j], mask=m)
                        pos = _iota() + (c * CH + base + j * 16)
                        plsc.store_compressed(
                            fifo_p.at[pl.ds(cnt2, 16)], pos, mask=m)
                        cnt2 = cnt2 + _popcnt(m)

                    def dcond(s):
                        rd, th = s
                        return rd + 16 <= cnt2

                    def dbody(s):
                        rd, th = s
                        cv = fifo_v[pl.ds(rd, 16)]
                        cp = fifo_p[pl.ds(rd, 16)]
                        bubble16(cv, cp)
                        return (rd + 16, _minlane(bufv[3]))

                    rd, theta2 = lax.while_loop(
                        dcond, dbody, (jnp.int32(0), theta))
                    n = cnt2 - rd
                    mres = _iota() < _i16(n)
                    rv = plsc.load_expanded(
                        fifo_v.at[pl.ds(rd, 16)], mask=mres)
                    rp = plsc.load_expanded(
                        fifo_p.at[pl.ds(rd, 16)], mask=mres)
                    plsc.store_compressed(
                        fifo_v.at[pl.ds(0, 16)], rv, mask=mres)
                    plsc.store_compressed(
                        fifo_p.at[pl.ds(0, 16)], rp, mask=mres)
                    return (n, theta2)

                return lax.cond(any_hit, trig, lambda carry: carry,
                                (cnt, theta))

            return lax.fori_loop(0, GROUPS, group_body, (cnt, theta))

        carry0 = (jnp.int32(0), jnp.float32(NEG))
        carry0 = chunk_body(0, carry0, 0, dma_sem0, dma_sem1)
        cnt, theta = chunk_body(1, carry0, 1, dma_sem1, dma_sem0)

        if True:
            mres = _iota() < _i16(cnt)
            cv = plsc.load_expanded(fifo_v.at[pl.ds(0, 16)], mask=mres)
            cp = plsc.load_expanded(fifo_p.at[pl.ds(0, 16)], mask=mres)
            bubble16(jnp.where(mres, cv, _f16(NEG)),
                     jnp.where(mres, cp, _i16(0)))

        def ext_body(k, carry):
            dv, ov, oi = carry
            W = plsc.load_gather(bufv, [dv, _iota()])
            Wi = plsc.load_gather(bufi, [dv, _iota()])
            packed = Wi * 16 + _iota()
            sv, sp = plsc.sort_key_val(W, packed, descending=True)
            p0 = _s(sp, 0)
            v0 = _f16(_s(sv, 0))
            tok = _i16(lax.shift_right_logical(p0, 4))
            l0 = _i16(lax.rem(p0, 16))
            dv = jnp.minimum(
                dv + jnp.where(_iota() == l0, _i16(1), _i16(0)),
                _i16(DEPTH - 1))
            ov = [jnp.where(_iota() == _i16(k - 16 * j), v0, ov[j])
                  for j in range(4)]
            oi = [jnp.where(_iota() == _i16(k - 16 * j), tok, oi[j])
                  for j in range(4)]
            return (dv, ov, oi)

        _, xs, ids = lax.fori_loop(
            0, K, ext_body,
            (_i16(0), [_f16(NEG)] * 4, [_i16(0)] * 4))

        tv_ = plsc.load_gather(scal_v, [_i16(r)])
        tkv = plsc.load_gather(scal_v, [_i16(RPW + r)])
        tpv = plsc.load_gather(scal_v, [_i16(2 * RPW + r)])
        mpv = plsc.load_gather(scal_v, [_i16(3 * RPW + r)])
        inv_t = _f16(1.0) / tv_
        pos = [_iota() + 16 * j for j in range(4)]
        posf = [p.astype(jnp.float32) for p in pos]
        km = [pf < tkv for pf in posf]
        masked = [jnp.where(km[j], xs[j] * inv_t, _f16(NEG))
                  for j in range(4)]
        m0v = _f16(_s(masked[0], 0))
        e = [jnp.exp(masked[j] - m0v) for j in range(4)]
        z = _vsum_f(e[0] + e[1] + e[2] + e[3])
        invz = _f16(1.0) / z
        pr = [e[j] * invz for j in range(4)]
        carry = _f16(0.0)
        excl = []
        for j in range(4):
            incl = _prefix_f(pr[j])
            excl.append(incl - pr[j] + carry)
            carry = carry + _f16(_s(incl, 15))
        pm = [excl[j] < tpv for j in range(4)]
        thr = mpv * _f16(_s(pr[0], 0))
        mm = [pr[j] >= thr for j in range(4)]
        fin = [(km[j] & pm[j] & mm[j]) | (pos[j] == _i16(0))
               for j in range(4)]
        mpr = [jnp.where(fin[j], pr[j], _f16(0.0)) for j in range(4)]
        zn = _vsum_f(mpr[0] + mpr[1] + mpr[2] + mpr[3])
        invzn = _f16(1.0) / zn
        for j in range(4):
            norm_st[pl.ds(r * K + 16 * j, 16)] = mpr[j] * invzn
        gs = [g_v[pl.ds(r * K + 16 * j, 16)] for j in range(4)]
        sc = [jnp.where(fin[j], masked[j] + gs[j], _f16(NEG))
              for j in range(4)]
        sv = jnp.maximum(jnp.maximum(sc[0], sc[1]),
                         jnp.maximum(sc[2], sc[3]))
        bms = _f16(_maxlane(sv))
        ch = [jnp.where(sc[j] == bms, pos[j], _i16(K)) for j in range(4)]
        cv2 = jnp.minimum(jnp.minimum(ch[0], ch[1]),
                          jnp.minimum(ch[2], ch[3]))
        sk, _sv2 = plsc.sort_key_val(cv2, _iota(), descending=False)
        choice = _i16(_s(sk, 0))
        nsel = [jnp.where(pos[j] == choice, ids[j], _i16(0))
                for j in range(4)]
        nid_spl = _vsum_i(nsel[0] + nsel[1] + nsel[2] + nsel[3])
        return jnp.where(_iota() == _i16(r), nid_spl, nid_vec)

    nid_vec_out = lax.fori_loop(0, RPW, row_body, _i16(0))
    nid_st[...] = nid_vec_out
    pltpu.sync_copy(norm_st, norm_hbm.at[pl.ds(wid * RPW * K, RPW * K)])
    pltpu.sync_copy(nid_st, nid_hbm.at[wid])


@functools.partial(jax.jit, static_argnums=())
def _sc_call(logits1d, scal, g):
    mesh = plsc.VectorSubcoreMesh(core_axis_name="c", subcore_axis_name="s")
    f = pl.kernel(
        _sc_body,
        mesh=mesh,
        compiler_params=pltpu.CompilerParams(needs_layout_passes=False),
        out_type=(
            jax.ShapeDtypeStruct((B * K,), jnp.float32),
            jax.ShapeDtypeStruct((NW, 16), jnp.int32),
        ),
        scratch_types=[
            pltpu.VMEM((16,), jnp.float32),
            pltpu.VMEM((RPW * K,), jnp.float32),
            pltpu.VMEM((2 * CH,), jnp.float32),
            pltpu.VMEM((FCAP,), jnp.float32),
            pltpu.VMEM((FCAP,), jnp.int32),
            pltpu.VMEM((DEPTH, 16), jnp.float32),
            pltpu.VMEM((DEPTH, 16), jnp.int32),
            pltpu.VMEM((16,), jnp.float32),
            pltpu.VMEM((16,), jnp.int32),
            pltpu.VMEM((RPW * K,), jnp.float32),
            pltpu.VMEM((16,), jnp.int32),
            pltpu.SemaphoreType.DMA,
            pltpu.SemaphoreType.DMA,
        ],
    )
    return f(logits1d, scal, g)


def kernel(logits, temperature, top_k, top_p, min_p):
    logits1d = logits.reshape(B * V)
    scal = jnp.concatenate(
        [temperature.reshape(NW, RPW),
         top_k.astype(jnp.float32).reshape(NW, RPW),
         top_p.reshape(NW, RPW),
         min_p.reshape(NW, RPW)], axis=1)
    g = jax.random.gumbel(
        jax.random.key(42), (B, K), jnp.float32).reshape(NW, RPW * K)
    norm1d, nid2 = _sc_call(logits1d, scal, g)
    return nid2[:, :RPW].reshape(B), norm1d.reshape(B, K)
